# baseline (speedup 1.0000x reference)
import functools

import jax
import jax.numpy as jnp
from jax import lax
from jax.experimental import pallas as pl
from jax.experimental.pallas import tpu as pltpu
from jax.experimental.pallas import tpu_sc as plsc

D = 32
U = 16
SEQ = 200
S0, S1 = 104, 96
NBUF = 8
_TR_BC = 32768


def _s_table_body(x_ref, w_ref, o_ref):
    x = x_ref[...]
    s = lax.dot_general(x * x, w_ref[...], (((0,), (0,)), ((), ())),
                        preferred_element_type=jnp.float32)
    s3 = s.reshape(_TR_BC // 8, 8, U)
    o_ref[...] = jnp.concatenate([s3[:, e, :] for e in range(8)], axis=1)


def _make_s_table(emb_t, w):
    vocab = emb_t.shape[1]
    grid = pl.cdiv(vocab, _TR_BC)
    return pl.pallas_call(
        _s_table_body,
        grid=(grid,),
        compiler_params=pltpu.CompilerParams(fuse_transposed_lhs_in_matmul=True),
        in_specs=[pl.BlockSpec((D, _TR_BC), lambda i: (0, i)),
                  pl.BlockSpec((D, U), lambda i: (0, 0))],
        out_specs=pl.BlockSpec((_TR_BC // 8, 8 * U), lambda i: (i, 0)),
        out_shape=jax.ShapeDtypeStruct((vocab // 8, 8 * U), jnp.float32),
    )(emb_t, w)


def _make_sc_kernel(batch):
    info = plsc.get_sparse_core_info()
    nw = info.num_cores * info.num_subcores
    assert batch % nw == 0
    rows_w = batch // nw
    assert rows_w % NBUF == 0

    mesh = plsc.VectorSubcoreMesh(core_axis_name="c", subcore_axis_name="s")

    @functools.partial(
        pl.kernel,
        mesh=mesh,
        compiler_params=pltpu.CompilerParams(use_tc_tiling_on_sc=False),
        out_type=jax.ShapeDtypeStruct((batch, U), jnp.float32),
        scratch_types=[
            pltpu.VMEM((rows_w, SEQ), jnp.int32),
            pltpu.VMEM((NBUF, SEQ, U), jnp.float32),
            pltpu.VMEM((rows_w, U), jnp.float32),
            pltpu.SemaphoreType.DMA((NBUF,)),
        ],
    )
    def sc_kernel(ids_hbm, s_hbm, out_hbm, idx_v, rows_v, out_v, sems):
        wid = lax.axis_index("s") * info.num_cores + lax.axis_index("c")
        base = wid * rows_w

        pltpu.sync_copy(ids_hbm.at[pl.ds(base, rows_w)], idx_v)

        def gather(r, b):
            c0 = pltpu.make_async_copy(
                s_hbm.at[idx_v.at[r, pl.ds(0, S0)]],
                rows_v.at[b, pl.ds(0, S0)], sems.at[b])
            c1 = pltpu.make_async_copy(
                s_hbm.at[idx_v.at[r, pl.ds(S0, S1)]],
                rows_v.at[b, pl.ds(S0, S1)], sems.at[b])
            return c0, c1

        for b in range(NBUF):
            c0, c1 = gather(b, b)
            c0.start()
            c1.start()

        def outer(g, carry):
            for b in range(NBUF):
                r = g * NBUF + b
                c0, c1 = gather(r, b)
                c0.wait()
                c1.wait()

                def tok(t, acc):
                    a0, a1, a2, a3 = acc
                    t8 = t * 8
                    a0 = a0 + rows_v[b, t8, pl.ds(0, U)]
                    a1 = a1 + rows_v[b, t8 + 1, pl.ds(0, U)]
                    a2 = a2 + rows_v[b, t8 + 2, pl.ds(0, U)]
                    a3 = a3 + rows_v[b, t8 + 3, pl.ds(0, U)]
                    a0 = a0 + rows_v[b, t8 + 4, pl.ds(0, U)]
                    a1 = a1 + rows_v[b, t8 + 5, pl.ds(0, U)]
                    a2 = a2 + rows_v[b, t8 + 6, pl.ds(0, U)]
                    a3 = a3 + rows_v[b, t8 + 7, pl.ds(0, U)]
                    return (a0, a1, a2, a3)

                zero = jnp.zeros((U,), jnp.float32)
                a0, a1, a2, a3 = lax.fori_loop(
                    0, SEQ // 8, tok, (zero, zero, zero, zero), unroll=5)

                @pl.when(r + NBUF < rows_w)
                def _():
                    n0, n1 = gather(r + NBUF, b)
                    n0.start()
                    n1.start()

                out_v[r, pl.ds(0, U)] = (a0 + a1) + (a2 + a3)
            return carry

        lax.fori_loop(0, rows_w // NBUF, outer, 0)
        pltpu.sync_copy(out_v, out_hbm.at[pl.ds(base, rows_w)])

    return sc_kernel


def kernel(input_ids, attention_mask, embedding, kernel):
    del attention_mask
    batch = input_ids.shape[0]
    vocab = embedding.shape[0]
    ids = input_ids.astype(jnp.int32)
    s_tab = _make_s_table(jnp.swapaxes(embedding, 0, 1), kernel)
    return _make_sc_kernel(batch)(ids, s_tab.reshape(vocab, U))


# speedup vs baseline: 1.0013x; 1.0013x over previous
import functools

import jax
import jax.numpy as jnp
from jax import lax
from jax.experimental import pallas as pl
from jax.experimental.pallas import tpu as pltpu
from jax.experimental.pallas import tpu_sc as plsc

D = 32
U = 16
SEQ = 200
S0, S1 = 104, 96
NBUF = 8
_TR_BC = 32768


def _s_table_body(x_ref, w_ref, o_ref):
    x = x_ref[...]
    s = lax.dot_general(x * x, w_ref[...], (((0,), (0,)), ((), ())),
                        preferred_element_type=jnp.float32)
    s3 = s.reshape(_TR_BC // 8, 8, U)
    o_ref[...] = jnp.concatenate([s3[:, e, :] for e in range(8)], axis=1)


def _make_s_table(emb_t, w):
    vocab = emb_t.shape[1]
    grid = pl.cdiv(vocab, _TR_BC)
    return pl.pallas_call(
        _s_table_body,
        grid=(grid,),
        compiler_params=pltpu.CompilerParams(fuse_transposed_lhs_in_matmul=True),
        in_specs=[pl.BlockSpec((D, _TR_BC), lambda i: (0, i)),
                  pl.BlockSpec((D, U), lambda i: (0, 0))],
        out_specs=pl.BlockSpec((_TR_BC // 8, 8 * U), lambda i: (i, 0)),
        out_shape=jax.ShapeDtypeStruct((vocab // 8, 8 * U), jnp.float32),
    )(emb_t, w)


def _make_sc_kernel(batch):
    info = plsc.get_sparse_core_info()
    nw = info.num_cores * info.num_subcores
    assert batch % nw == 0
    rows_w = batch // nw
    assert rows_w % NBUF == 0

    mesh = plsc.VectorSubcoreMesh(core_axis_name="c", subcore_axis_name="s")

    @functools.partial(
        pl.kernel,
        mesh=mesh,
        compiler_params=pltpu.CompilerParams(use_tc_tiling_on_sc=False),
        out_type=jax.ShapeDtypeStruct((batch, U), jnp.float32),
        scratch_types=[
            pltpu.VMEM((rows_w, SEQ), jnp.int32),
            pltpu.VMEM((NBUF, SEQ, U), jnp.float32),
            pltpu.VMEM((rows_w, U), jnp.float32),
            pltpu.SemaphoreType.DMA((NBUF,)),
        ],
    )
    def sc_kernel(ids_hbm, s_hbm, out_hbm, idx_v, rows_v, out_v, sems):
        wid = lax.axis_index("s") * info.num_cores + lax.axis_index("c")
        base = wid * rows_w

        pltpu.sync_copy(ids_hbm.at[pl.ds(base, rows_w)], idx_v)

        def gather(r, b):
            c0 = pltpu.make_async_copy(
                s_hbm.at[idx_v.at[r, pl.ds(0, S0)]],
                rows_v.at[b, pl.ds(0, S0)], sems.at[b])
            c1 = pltpu.make_async_copy(
                s_hbm.at[idx_v.at[r, pl.ds(S0, S1)]],
                rows_v.at[b, pl.ds(S0, S1)], sems.at[b])
            return c0, c1

        for b in range(NBUF):
            c0, c1 = gather(b, b)
            c0.start()
            c1.start()

        def outer(g, carry):
            for b in range(NBUF):
                r = g * NBUF + b
                c0, c1 = gather(r, b)
                c0.wait()
                c1.wait()

                def tok(t, acc):
                    a0, a1, a2, a3 = acc
                    t8 = t * 8
                    a0 = a0 + rows_v[b, t8, pl.ds(0, U)]
                    a1 = a1 + rows_v[b, t8 + 1, pl.ds(0, U)]
                    a2 = a2 + rows_v[b, t8 + 2, pl.ds(0, U)]
                    a3 = a3 + rows_v[b, t8 + 3, pl.ds(0, U)]
                    a0 = a0 + rows_v[b, t8 + 4, pl.ds(0, U)]
                    a1 = a1 + rows_v[b, t8 + 5, pl.ds(0, U)]
                    a2 = a2 + rows_v[b, t8 + 6, pl.ds(0, U)]
                    a3 = a3 + rows_v[b, t8 + 7, pl.ds(0, U)]
                    return (a0, a1, a2, a3)

                zero = jnp.zeros((U,), jnp.float32)
                a0, a1, a2, a3 = lax.fori_loop(
                    0, SEQ // 8, tok, (zero, zero, zero, zero))

                @pl.when(r + NBUF < rows_w)
                def _():
                    n0, n1 = gather(r + NBUF, b)
                    n0.start()
                    n1.start()

                out_v[r, pl.ds(0, U)] = (a0 + a1) + (a2 + a3)
            return carry

        lax.fori_loop(0, rows_w // NBUF, outer, 0)
        pltpu.sync_copy(out_v, out_hbm.at[pl.ds(base, rows_w)])

    return sc_kernel


def kernel(input_ids, attention_mask, embedding, kernel):
    del attention_mask
    batch = input_ids.shape[0]
    vocab = embedding.shape[0]
    ids = input_ids.astype(jnp.int32)
    s_tab = _make_s_table(jnp.swapaxes(embedding, 0, 1), kernel)
    return _make_sc_kernel(batch)(ids, s_tab.reshape(vocab, U))
